# baseline (speedup 1.0000x reference)
import functools

import jax
import jax.numpy as jnp
from jax import lax
from jax.experimental import pallas as pl
from jax.experimental.pallas import tpu as pltpu
from jax.experimental.pallas import tpu_sc as plsc

N = 10000
E = 320000
D = 128
NP = 10240
NC = 2
NS = 16
NT = NC * NS
RPT = NP // NT * NC
K = 128
NBT = 80
EP = NT * NBT * K
TRASH = N + 64
B = 1024

_mesh = plsc.VectorSubcoreMesh(core_axis_name="c", subcore_axis_name="s")


def _stage_idx(dst_ref, src_ref, b_i):
    for j in range(K // 16):
        dst_ref[pl.ds(j * 16, 16)] = src_ref[b_i, pl.ds(j * 16, 16)]


@functools.partial(
    pl.kernel,
    out_type=jax.ShapeDtypeStruct((NC, NP), jnp.float32),
    mesh=_mesh,
    scratch_types=[
        pltpu.VMEM((NBT, K), jnp.int32),
        pltpu.VMEM((K,), jnp.int32),
        pltpu.VMEM((K,), jnp.int32),
        pltpu.VMEM((K,), jnp.float32),
        pltpu.VMEM((RPT,), jnp.float32),
        pltpu.VMEM_SHARED((NP,), jnp.float32),
        pltpu.SemaphoreType.DMA,
        pltpu.SemaphoreType.DMA,
    ],
)
def _sc_degree(dstm_hbm, out_hbm, dst_all, cur0, cur1, ones_v, stage_v,
               deg_sh, sem0, sem1):
    c = lax.axis_index("c")
    s = lax.axis_index("s")
    for j in range(K // 16):
        ones_v[pl.ds(j * 16, 16)] = jnp.ones((16,), jnp.float32)

    def zb(r, carry):
        stage_v[pl.ds(r * 16, 16)] = jnp.zeros((16,), jnp.float32)
        return carry

    lax.fori_loop(0, RPT // 16, zb, 0)
    r0 = s * RPT
    pltpu.sync_copy(stage_v, deg_sh.at[pl.ds(r0, RPT)])
    plsc.subcore_barrier()

    nb0 = (c * NS + s) * NBT
    pltpu.sync_copy(dstm_hbm.at[pl.ds(nb0, NBT)], dst_all)

    _stage_idx(cur0, dst_all, 0)
    pltpu.async_copy(ones_v, deg_sh.at[cur0], sem0, add=True)
    _stage_idx(cur1, dst_all, 1)
    pltpu.async_copy(ones_v, deg_sh.at[cur1], sem1, add=True)

    def fire(i, carry):
        b0 = 2 * i
        pltpu.make_async_copy(ones_v, deg_sh.at[cur0], sem0).wait()
        _stage_idx(cur0, dst_all, b0 + 2)
        pltpu.async_copy(ones_v, deg_sh.at[cur0], sem0, add=True)
        pltpu.make_async_copy(ones_v, deg_sh.at[cur1], sem1).wait()
        _stage_idx(cur1, dst_all, b0 + 3)
        pltpu.async_copy(ones_v, deg_sh.at[cur1], sem1, add=True)
        return carry

    lax.fori_loop(0, NBT // 2 - 1, fire, 0)
    pltpu.make_async_copy(ones_v, deg_sh.at[cur0], sem0).wait()
    pltpu.make_async_copy(ones_v, deg_sh.at[cur1], sem1).wait()
    plsc.subcore_barrier()
    pltpu.sync_copy(deg_sh.at[pl.ds(r0, RPT)], stage_v)
    pltpu.sync_copy(stage_v, out_hbm.at[c, pl.ds(r0, RPT)])


KA = 64
NBA = -(-(E // NT) // KA)
EBR = NT * NBA
EA = EBR * KA


def _stage_idx_k(dst_ref, src_ref, b_i):
    for j in range(KA // 16):
        dst_ref[pl.ds(j * 16, 16)] = src_ref[b_i, pl.ds(j * 16, 16)]


@functools.partial(
    pl.kernel,
    out_type=jax.ShapeDtypeStruct((NC, NP, D), jnp.float32),
    mesh=_mesh,
    scratch_types=[
        pltpu.VMEM((2, KA), jnp.int32),
        pltpu.VMEM((2, KA), jnp.int32),
        pltpu.VMEM((KA,), jnp.int32),
        pltpu.VMEM((KA,), jnp.int32),
        pltpu.VMEM((KA, D), jnp.float32),
        pltpu.VMEM_SHARED((NP, D), jnp.float32),
        pltpu.SemaphoreType.DMA,
        pltpu.SemaphoreType.DMA,
        pltpu.SemaphoreType.DMA,
    ],
)
def _sc_edge_agg(h_hbm, ebm_hbm, out_hbm, ib0, ib1, scur, dcur, rows_v,
                 agg_sh, isem0, isem1, gsem):
    c = lax.axis_index("c")
    s = lax.axis_index("s")

    def zb(r, carry):
        for j in range(D // 16):
            rows_v[r, pl.ds(j * 16, 16)] = jnp.zeros((16,), jnp.float32)
        return carry

    lax.fori_loop(0, KA, zb, 0)
    r0 = s * RPT
    for j in range(RPT // KA):
        pltpu.sync_copy(rows_v, agg_sh.at[pl.ds(r0 + j * KA, KA)])
    plsc.subcore_barrier()

    base = (c * NS + s) * NBA

    def prefetch(ib, sem, b_i):
        row = jnp.minimum(base + b_i, EBR - 1)
        pltpu.async_copy(ebm_hbm.at[row], ib, sem)

    def do_batch(ib, sem, b_i, nxt):
        pltpu.make_async_copy(ebm_hbm.at[0], ib, sem).wait()
        _stage_idx_k(scur, ib, 0)
        _stage_idx_k(dcur, ib, 1)
        prefetch(ib, sem, nxt)
        pltpu.async_copy(h_hbm.at[scur], rows_v, gsem).wait()
        pltpu.sync_copy(rows_v, agg_sh.at[dcur], add=True)

    prefetch(ib0, isem0, 0)
    prefetch(ib1, isem1, 1)

    def body(i, carry):
        b0 = 2 * i
        do_batch(ib0, isem0, b0, b0 + 2)
        do_batch(ib1, isem1, b0 + 1, b0 + 3)
        return carry

    lax.fori_loop(0, NBA // 2, body, 0)
    do_batch(ib0, isem0, NBA - 1, NBA - 1)
    pltpu.make_async_copy(ebm_hbm.at[0], ib0, isem0).wait()
    pltpu.make_async_copy(ebm_hbm.at[0], ib1, isem1).wait()

    plsc.subcore_barrier()
    for j in range(RPT // KA):
        pltpu.sync_copy(agg_sh.at[pl.ds(r0 + j * KA, KA)], rows_v)
        pltpu.sync_copy(rows_v, out_hbm.at[c, pl.ds(r0 + j * KA, KA)])


def _tc1_body(x_ref, w_ref, degp_ref, h1p_ref, dinv_ref):
    d = degp_ref[0] + degp_ref[1] + 1.0
    dinv = lax.rsqrt(jnp.maximum(d, 1e-12))
    dinv_ref[...] = dinv
    h1p_ref[...] = dinv * jnp.dot(x_ref[...], w_ref[...],
                                  preferred_element_type=jnp.float32)


def _tc1(xp, W1, degp3):
    return pl.pallas_call(
        _tc1_body,
        grid=(NP // B,),
        in_specs=[
            pl.BlockSpec((B, D), lambda i: (i, 0)),
            pl.BlockSpec((D, D), lambda i: (0, 0)),
            pl.BlockSpec((NC, B, 1), lambda i: (0, i, 0)),
        ],
        out_specs=[
            pl.BlockSpec((B, D), lambda i: (i, 0)),
            pl.BlockSpec((B, 1), lambda i: (i, 0)),
        ],
        out_shape=[
            jax.ShapeDtypeStruct((NP, D), jnp.float32),
            jax.ShapeDtypeStruct((NP, 1), jnp.float32),
        ],
    )(xp, W1, degp3)


def _tc2_body(aggp_ref, hp_ref, dinv_ref, b_ref, aw_ref, ab_ref, w2_ref,
              out_ref):
    dinv = dinv_ref[...]
    t = dinv * (aggp_ref[0] + aggp_ref[1] + hp_ref[...]) + b_ref[...]
    t = jnp.maximum(t, 0.0)
    s = jax.nn.sigmoid(jnp.dot(t, aw_ref[...],
                               preferred_element_type=jnp.float32) + ab_ref[...])
    out_ref[...] = dinv * jnp.dot(t * s, w2_ref[...],
                                  preferred_element_type=jnp.float32)


def _tc2(aggp, hp, dinv, b1r, a1w, a1br, W2):
    return pl.pallas_call(
        _tc2_body,
        grid=(NP // B,),
        in_specs=[
            pl.BlockSpec((NC, B, D), lambda i: (0, i, 0)),
            pl.BlockSpec((B, D), lambda i: (i, 0)),
            pl.BlockSpec((B, 1), lambda i: (i, 0)),
            pl.BlockSpec((1, D), lambda i: (0, 0)),
            pl.BlockSpec((D, 1), lambda i: (0, 0)),
            pl.BlockSpec((1, 1), lambda i: (0, 0)),
            pl.BlockSpec((D, D), lambda i: (0, 0)),
        ],
        out_specs=pl.BlockSpec((B, D), lambda i: (i, 0)),
        out_shape=jax.ShapeDtypeStruct((NP, D), jnp.float32),
    )(aggp, hp, dinv, b1r, a1w, a1br, W2)


def _tc3_body(aggp_ref, hp_ref, dinv_ref, b_ref, aw_ref, ab_ref, out_ref):
    t = dinv_ref[...] * (aggp_ref[0] + aggp_ref[1] + hp_ref[...]) + b_ref[...]
    s = jax.nn.sigmoid(jnp.dot(t, aw_ref[...],
                               preferred_element_type=jnp.float32) + ab_ref[...])
    out_ref[...] = t * s


def _tc3(aggp, hp, dinv, b2r, a2w, a2br):
    return pl.pallas_call(
        _tc3_body,
        grid=(NP // B,),
        in_specs=[
            pl.BlockSpec((NC, B, D), lambda i: (0, i, 0)),
            pl.BlockSpec((B, D), lambda i: (i, 0)),
            pl.BlockSpec((B, 1), lambda i: (i, 0)),
            pl.BlockSpec((1, D), lambda i: (0, 0)),
            pl.BlockSpec((D, 1), lambda i: (0, 0)),
            pl.BlockSpec((1, 1), lambda i: (0, 0)),
        ],
        out_specs=pl.BlockSpec((B, D), lambda i: (i, 0)),
        out_shape=jax.ShapeDtypeStruct((NP, D), jnp.float32),
    )(aggp, hp, dinv, b2r, a2w, a2br)


def kernel(x, edge_index, W1, b1, W2, b2, a1w, a1b, a2w, a2b):
    src = edge_index[0]
    dst = edge_index[1]
    pad = EP - E
    srcm = jnp.concatenate(
        [src, jnp.zeros((pad,), jnp.int32)]).reshape(EP // K, K)
    dstm = jnp.concatenate(
        [dst, jnp.full((pad,), TRASH, jnp.int32)]).reshape(EP // K, K)
    xp = jnp.pad(x, ((0, NP - N), (0, 0)))

    pada = EA - E
    srca = jnp.concatenate([src, jnp.zeros((pada,), jnp.int32)])
    dsta = jnp.concatenate([dst, jnp.full((pada,), TRASH, jnp.int32)])
    ebm = jnp.stack([srca.reshape(EBR, KA), dsta.reshape(EBR, KA)], axis=1)

    degp = _sc_degree(dstm)
    degp3 = degp.reshape(NC, NP, 1)
    h1p, dinv = _tc1(xp, W1, degp3)
    agg1 = _sc_edge_agg(h1p, ebm)
    h2p = _tc2(agg1, h1p, dinv, b1.reshape(1, D), a1w, a1b.reshape(1, 1), W2)
    agg2 = _sc_edge_agg(h2p, ebm)
    out = _tc3(agg2, h2p, dinv, b2.reshape(1, D), a2w, a2b.reshape(1, 1))
    return out[:N]


# speedup vs baseline: 2.0250x; 2.0250x over previous
import functools

import jax
import jax.numpy as jnp
from jax import lax
from jax.experimental import pallas as pl
from jax.experimental.pallas import tpu as pltpu
from jax.experimental.pallas import tpu_sc as plsc

N = 10000
E = 320000
D = 128
NP = 10240
NC = 2
NS = 16
NT = NC * NS
RPT = NP // NT * NC
K = 128
NBT = 80
EP = NT * NBT * K
TRASH = N + 64
B = 1024

_mesh = plsc.VectorSubcoreMesh(core_axis_name="c", subcore_axis_name="s")


def _stage_idx(dst_ref, src_ref, b_i):
    for j in range(K // 16):
        dst_ref[pl.ds(j * 16, 16)] = src_ref[b_i, pl.ds(j * 16, 16)]


@functools.partial(
    pl.kernel,
    out_type=jax.ShapeDtypeStruct((NC, NP), jnp.float32),
    mesh=_mesh,
    scratch_types=[
        pltpu.VMEM((NBT, K), jnp.int32),
        pltpu.VMEM((K,), jnp.int32),
        pltpu.VMEM((K,), jnp.int32),
        pltpu.VMEM((K,), jnp.float32),
        pltpu.VMEM((RPT,), jnp.float32),
        pltpu.VMEM_SHARED((NP,), jnp.float32),
        pltpu.SemaphoreType.DMA,
        pltpu.SemaphoreType.DMA,
    ],
)
def _sc_degree(dstm_hbm, out_hbm, dst_all, cur0, cur1, ones_v, stage_v,
               deg_sh, sem0, sem1):
    c = lax.axis_index("c")
    s = lax.axis_index("s")
    for j in range(K // 16):
        ones_v[pl.ds(j * 16, 16)] = jnp.ones((16,), jnp.float32)

    def zb(r, carry):
        stage_v[pl.ds(r * 16, 16)] = jnp.zeros((16,), jnp.float32)
        return carry

    lax.fori_loop(0, RPT // 16, zb, 0)
    r0 = s * RPT
    pltpu.sync_copy(stage_v, deg_sh.at[pl.ds(r0, RPT)])
    plsc.subcore_barrier()

    nb0 = (c * NS + s) * NBT
    pltpu.sync_copy(dstm_hbm.at[pl.ds(nb0, NBT)], dst_all)

    _stage_idx(cur0, dst_all, 0)
    pltpu.async_copy(ones_v, deg_sh.at[cur0], sem0, add=True)
    _stage_idx(cur1, dst_all, 1)
    pltpu.async_copy(ones_v, deg_sh.at[cur1], sem1, add=True)

    def fire(i, carry):
        b0 = 2 * i
        pltpu.make_async_copy(ones_v, deg_sh.at[cur0], sem0).wait()
        _stage_idx(cur0, dst_all, b0 + 2)
        pltpu.async_copy(ones_v, deg_sh.at[cur0], sem0, add=True)
        pltpu.make_async_copy(ones_v, deg_sh.at[cur1], sem1).wait()
        _stage_idx(cur1, dst_all, b0 + 3)
        pltpu.async_copy(ones_v, deg_sh.at[cur1], sem1, add=True)
        return carry

    lax.fori_loop(0, NBT // 2 - 1, fire, 0)
    pltpu.make_async_copy(ones_v, deg_sh.at[cur0], sem0).wait()
    pltpu.make_async_copy(ones_v, deg_sh.at[cur1], sem1).wait()
    plsc.subcore_barrier()
    pltpu.sync_copy(deg_sh.at[pl.ds(r0, RPT)], stage_v)
    pltpu.sync_copy(stage_v, out_hbm.at[c, pl.ds(r0, RPT)])


KA = 80
NBA = -(-(E // NT) // KA)
EBR = NT * NBA
EA = EBR * KA


def _stage_idx_k(dst_ref, src_ref, b_i):
    for j in range(KA // 16):
        dst_ref[pl.ds(j * 16, 16)] = src_ref[b_i, pl.ds(j * 16, 16)]


@functools.partial(
    pl.kernel,
    out_type=jax.ShapeDtypeStruct((NC, NP, D), jnp.float32),
    mesh=_mesh,
    scratch_types=[
        pltpu.VMEM((2, KA), jnp.int32),
        pltpu.VMEM((2, KA), jnp.int32),
        pltpu.VMEM((KA,), jnp.int32),
        pltpu.VMEM((KA,), jnp.int32),
        pltpu.VMEM((KA,), jnp.int32),
        pltpu.VMEM((KA,), jnp.int32),
        pltpu.VMEM((KA, D), jnp.float32),
        pltpu.VMEM((KA, D), jnp.float32),
        pltpu.VMEM_SHARED((NP, D), jnp.float32),
        pltpu.SemaphoreType.DMA,
        pltpu.SemaphoreType.DMA,
        pltpu.SemaphoreType.DMA,
        pltpu.SemaphoreType.DMA,
    ],
)
def _sc_edge_agg(h_hbm, ebm_hbm, out_hbm, ib0, ib1, scur0, scur1,
                 dcur0, dcur1, rows0, rows1, agg_sh, isem0, isem1,
                 gsem0, gsem1):
    c = lax.axis_index("c")
    s = lax.axis_index("s")

    def zb(r, carry):
        for j in range(D // 16):
            rows0[r, pl.ds(j * 16, 16)] = jnp.zeros((16,), jnp.float32)
        return carry

    lax.fori_loop(0, KA, zb, 0)
    r0 = s * RPT
    for j in range(RPT // KA):
        pltpu.sync_copy(rows0, agg_sh.at[pl.ds(r0 + j * KA, KA)])
    plsc.subcore_barrier()

    base = (c * NS + s) * NBA

    def prefetch(ib, sem, b_i):
        row = jnp.minimum(base + b_i, EBR - 1)
        pltpu.async_copy(ebm_hbm.at[row], ib, sem)

    def launch(ib, isem, scur, dcur, rows, gsem, nxt):
        pltpu.make_async_copy(ebm_hbm.at[0], ib, isem).wait()
        _stage_idx_k(scur, ib, 0)
        _stage_idx_k(dcur, ib, 1)
        prefetch(ib, isem, nxt)
        pltpu.async_copy(h_hbm.at[scur], rows, gsem)

    def settle(scur, dcur, rows, gsem):
        pltpu.make_async_copy(h_hbm.at[scur], rows, gsem).wait()
        pltpu.sync_copy(rows, agg_sh.at[dcur], add=True)

    prefetch(ib0, isem0, 0)
    prefetch(ib1, isem1, 1)
    launch(ib0, isem0, scur0, dcur0, rows0, gsem0, 2)
    launch(ib1, isem1, scur1, dcur1, rows1, gsem1, 3)

    def body(i, carry):
        b0 = 2 * i
        settle(scur0, dcur0, rows0, gsem0)
        launch(ib0, isem0, scur0, dcur0, rows0, gsem0, b0 + 4)
        settle(scur1, dcur1, rows1, gsem1)
        launch(ib1, isem1, scur1, dcur1, rows1, gsem1, b0 + 5)
        return carry

    lax.fori_loop(0, NBA // 2 - 1, body, 0)
    settle(scur0, dcur0, rows0, gsem0)
    launch(ib0, isem0, scur0, dcur0, rows0, gsem0, NBA - 1)
    settle(scur1, dcur1, rows1, gsem1)
    settle(scur0, dcur0, rows0, gsem0)
    pltpu.make_async_copy(ebm_hbm.at[0], ib0, isem0).wait()
    pltpu.make_async_copy(ebm_hbm.at[0], ib1, isem1).wait()

    plsc.subcore_barrier()
    for j in range(RPT // KA):
        pltpu.sync_copy(agg_sh.at[pl.ds(r0 + j * KA, KA)], rows0)
        pltpu.sync_copy(rows0, out_hbm.at[c, pl.ds(r0 + j * KA, KA)])


def _tc1_body(x_ref, w_ref, degp_ref, h1p_ref, dinv_ref):
    d = degp_ref[0] + degp_ref[1] + 1.0
    dinv = lax.rsqrt(jnp.maximum(d, 1e-12))
    dinv_ref[...] = dinv
    h1p_ref[...] = dinv * jnp.dot(x_ref[...], w_ref[...],
                                  preferred_element_type=jnp.float32)


def _tc1(xp, W1, degp3):
    return pl.pallas_call(
        _tc1_body,
        grid=(NP // B,),
        in_specs=[
            pl.BlockSpec((B, D), lambda i: (i, 0)),
            pl.BlockSpec((D, D), lambda i: (0, 0)),
            pl.BlockSpec((NC, B, 1), lambda i: (0, i, 0)),
        ],
        out_specs=[
            pl.BlockSpec((B, D), lambda i: (i, 0)),
            pl.BlockSpec((B, 1), lambda i: (i, 0)),
        ],
        out_shape=[
            jax.ShapeDtypeStruct((NP, D), jnp.float32),
            jax.ShapeDtypeStruct((NP, 1), jnp.float32),
        ],
    )(xp, W1, degp3)


def _tc2_body(aggp_ref, hp_ref, dinv_ref, b_ref, aw_ref, ab_ref, w2_ref,
              out_ref):
    dinv = dinv_ref[...]
    t = dinv * (aggp_ref[0] + aggp_ref[1] + hp_ref[...]) + b_ref[...]
    t = jnp.maximum(t, 0.0)
    s = jax.nn.sigmoid(jnp.dot(t, aw_ref[...],
                               preferred_element_type=jnp.float32) + ab_ref[...])
    out_ref[...] = dinv * jnp.dot(t * s, w2_ref[...],
                                  preferred_element_type=jnp.float32)


def _tc2(aggp, hp, dinv, b1r, a1w, a1br, W2):
    return pl.pallas_call(
        _tc2_body,
        grid=(NP // B,),
        in_specs=[
            pl.BlockSpec((NC, B, D), lambda i: (0, i, 0)),
            pl.BlockSpec((B, D), lambda i: (i, 0)),
            pl.BlockSpec((B, 1), lambda i: (i, 0)),
            pl.BlockSpec((1, D), lambda i: (0, 0)),
            pl.BlockSpec((D, 1), lambda i: (0, 0)),
            pl.BlockSpec((1, 1), lambda i: (0, 0)),
            pl.BlockSpec((D, D), lambda i: (0, 0)),
        ],
        out_specs=pl.BlockSpec((B, D), lambda i: (i, 0)),
        out_shape=jax.ShapeDtypeStruct((NP, D), jnp.float32),
    )(aggp, hp, dinv, b1r, a1w, a1br, W2)


def _tc3_body(aggp_ref, hp_ref, dinv_ref, b_ref, aw_ref, ab_ref, out_ref):
    t = dinv_ref[...] * (aggp_ref[0] + aggp_ref[1] + hp_ref[...]) + b_ref[...]
    s = jax.nn.sigmoid(jnp.dot(t, aw_ref[...],
                               preferred_element_type=jnp.float32) + ab_ref[...])
    out_ref[...] = t * s


def _tc3(aggp, hp, dinv, b2r, a2w, a2br):
    return pl.pallas_call(
        _tc3_body,
        grid=(NP // B,),
        in_specs=[
            pl.BlockSpec((NC, B, D), lambda i: (0, i, 0)),
            pl.BlockSpec((B, D), lambda i: (i, 0)),
            pl.BlockSpec((B, 1), lambda i: (i, 0)),
            pl.BlockSpec((1, D), lambda i: (0, 0)),
            pl.BlockSpec((D, 1), lambda i: (0, 0)),
            pl.BlockSpec((1, 1), lambda i: (0, 0)),
        ],
        out_specs=pl.BlockSpec((B, D), lambda i: (i, 0)),
        out_shape=jax.ShapeDtypeStruct((NP, D), jnp.float32),
    )(aggp, hp, dinv, b2r, a2w, a2br)


def kernel(x, edge_index, W1, b1, W2, b2, a1w, a1b, a2w, a2b):
    src = edge_index[0]
    dst = edge_index[1]
    pad = EP - E
    srcm = jnp.concatenate(
        [src, jnp.zeros((pad,), jnp.int32)]).reshape(EP // K, K)
    dstm = jnp.concatenate(
        [dst, jnp.full((pad,), TRASH, jnp.int32)]).reshape(EP // K, K)
    xp = jnp.pad(x, ((0, NP - N), (0, 0)))

    pada = EA - E
    srca = jnp.concatenate([src, jnp.zeros((pada,), jnp.int32)])
    dsta = jnp.concatenate([dst, jnp.full((pada,), TRASH, jnp.int32)])
    ebm = jnp.stack([srca.reshape(EBR, KA), dsta.reshape(EBR, KA)], axis=1)

    degp = _sc_degree(dstm)
    degp3 = degp.reshape(NC, NP, 1)
    h1p, dinv = _tc1(xp, W1, degp3)
    agg1 = _sc_edge_agg(h1p, ebm)
    h2p = _tc2(agg1, h1p, dinv, b1.reshape(1, D), a1w, a1b.reshape(1, 1), W2)
    agg2 = _sc_edge_agg(h2p, ebm)
    out = _tc3(agg2, h2p, dinv, b2.reshape(1, D), a2w, a2b.reshape(1, 1))
    return out[:N]
